# Initial kernel scaffold; baseline (speedup 1.0000x reference)
import jax
import jax.numpy as jnp
from jax.experimental import pallas as pl


def kernel(x_lex, edge_index, batch, emb, W1, a1_src, a1_dst, b1, W2, a2_src, a2_dst, b2, Wc1, bc1, Wc2, bc2):
    raise NotImplementedError("write your pallas kernel here")


# speedup vs baseline: 12.9910x; 12.9910x over previous
import functools

import jax
import jax.numpy as jnp
from jax import lax
from jax.experimental import pallas as pl
from jax.experimental.pallas import tpu as pltpu
from jax.experimental.pallas import tpu_sc as plsc

N_NODES = 50000
N_EDGES = 800000
VOCAB = 100000
EMBED = 64
HID = 64
HEADS = 4
N_GRAPHS = 128

NC, NS, L = 2, 16, 16
NW = NC * NS

RB = 1024
NODE_PAD = 50176
GRID_R = NODE_PAD // RB

CHUNK = 64
EDGES_PER_TILE = 50176
EDGE_PAD = NS * EDGES_PER_TILE
N_CHUNKS = EDGES_PER_TILE // CHUNK

TW = 32
DW = 16
AW = 24

_EPS = 1e-16

_SC_PARAMS = pltpu.CompilerParams(use_tc_tiling_on_sc=False,
                                  needs_layout_passes=False)


def _emb_gather(emb, idx_pad):
    GW = 128
    n_win = NODE_PAD // GW
    per_w = -(-n_win // NW)
    mesh = plsc.VectorSubcoreMesh(core_axis_name="c", subcore_axis_name="s")

    @functools.partial(
        pl.kernel,
        out_type=jax.ShapeDtypeStruct((NODE_PAD, EMBED), jnp.float32),
        mesh=mesh,
        scratch_types=[
            pltpu.VMEM((GW,), jnp.int32),
            pltpu.VMEM((GW, EMBED), jnp.float32),
            pltpu.SemaphoreType.DMA,
        ],
        compiler_params=_SC_PARAMS,
    )
    def k(emb_hbm, idx_hbm, out_hbm, idx_v, rows_v, sem):
        wid = lax.axis_index("s") * NC + lax.axis_index("c")

        @pl.loop(0, per_w)
        def _(i):
            w = wid * per_w + i

            @pl.when(w < n_win)
            def _():
                base = w * GW
                pltpu.sync_copy(idx_hbm.at[pl.ds(base, GW)], idx_v)
                pltpu.async_copy(emb_hbm.at[idx_v], rows_v, sem).wait()
                pltpu.sync_copy(rows_v, out_hbm.at[pl.ds(base, GW)])

    return k(emb, idx_pad)


def _edge_pass(t_all, d_all, src2, dst2, dst_pad, zacc):
    mesh = plsc.VectorSubcoreMesh(core_axis_name="c", subcore_axis_name="s")
    zrows = NODE_PAD // NS

    @functools.partial(
        pl.kernel,
        out_type=jax.ShapeDtypeStruct((2 * NODE_PAD, AW), jnp.float32),
        mesh=mesh,
        scratch_types=[
            pltpu.VMEM((CHUNK,), jnp.int32),
            pltpu.VMEM((CHUNK,), jnp.int32),
            pltpu.VMEM((CHUNK,), jnp.int32),
            pltpu.VMEM((CHUNK, TW), jnp.float32),
            pltpu.VMEM((CHUNK, DW), jnp.float32),
            pltpu.VMEM((CHUNK, AW), jnp.float32),
            pltpu.VMEM_SHARED((NODE_PAD, AW), jnp.float32),
        ],
        compiler_params=_SC_PARAMS,
    )
    def k(t_hbm, d_hbm, src2_hbm, dst2_hbm, dst_hbm, z_hbm, acc_hbm,
          dsti_v, srco_v, dsto_v, trow_v, drow_v, blk_v, acc_sp):
        cid = lax.axis_index("c")
        sid = lax.axis_index("s")
        coff = cid * NODE_PAD

        pltpu.sync_copy(z_hbm.at[pl.ds(sid * zrows, zrows)],
                        acc_sp.at[pl.ds(sid * zrows, zrows)])
        plsc.subcore_barrier()

        @pl.loop(0, N_CHUNKS)
        def _chunk(ci):
            ebase = sid * EDGES_PER_TILE + ci * CHUNK
            pltpu.sync_copy(src2_hbm.at[cid, pl.ds(ebase, CHUNK)], srco_v)
            pltpu.sync_copy(dst2_hbm.at[cid, pl.ds(ebase, CHUNK)], dsto_v)
            pltpu.sync_copy(dst_hbm.at[pl.ds(ebase, CHUNK)], dsti_v)
            pltpu.sync_copy(t_hbm.at[srco_v], trow_v)
            pltpu.sync_copy(d_hbm.at[dsto_v], drow_v)
            for g in range(CHUNK // L):
                rows = lax.iota(jnp.int32, L) + (g * L)
                c16 = jnp.full((L,), 16, jnp.int32)
                c0 = jnp.zeros((L,), jnp.int32)
                s0 = plsc.load_gather(trow_v, [rows, c16]) + \
                     plsc.load_gather(drow_v, [rows, c0])
                ee = jnp.exp(jnp.maximum(s0, 0.2 * s0))
                plsc.store_scatter(blk_v, [rows, c16], ee)
                for j in range(L):
                    e = g * L + j
                    blk_v[e, pl.ds(0, L)] = trow_v[e, pl.ds(0, L)] * ee[j]
            pltpu.sync_copy(blk_v, acc_sp.at[dsti_v], add=True)

        plsc.subcore_barrier()
        obase = coff + sid * zrows
        pltpu.sync_copy(acc_sp.at[pl.ds(sid * zrows, zrows)],
                        acc_hbm.at[pl.ds(obase, zrows)])

    return k(t_all, d_all, src2, dst2, dst_pad, zacc)


def _pack1(x, W1, A1s, A1d):
    def body(x_ref, w_ref, as_ref, ad_ref, t0_ref, d0_ref, t1_ref, d1_ref):
        h = jnp.dot(x_ref[...], w_ref[...], preferred_element_type=jnp.float32)
        ss = jnp.dot(h, as_ref[...], preferred_element_type=jnp.float32)
        sd = jnp.dot(h, ad_ref[...], preferred_element_type=jnp.float32)
        zt = jnp.zeros((RB, TW - 17), jnp.float32)
        zd = jnp.zeros((RB, DW - 1), jnp.float32)
        for p, (t_ref, d_ref) in ((0, (t0_ref, d0_ref)), (1, (t1_ref, d1_ref))):
            for c in range(2):
                hd = 2 * c + p
                t_ref[c] = jnp.concatenate(
                    [h[:, hd * 16:(hd + 1) * 16], ss[:, hd:hd + 1], zt], axis=1)
                d_ref[c] = jnp.concatenate([sd[:, hd:hd + 1], zd], axis=1)

    return pl.pallas_call(
        body,
        grid=(GRID_R,),
        in_specs=[
            pl.BlockSpec((RB, EMBED), lambda i: (i, 0)),
            pl.BlockSpec((EMBED, HID), lambda i: (0, 0)),
            pl.BlockSpec((HID, HEADS), lambda i: (0, 0)),
            pl.BlockSpec((HID, HEADS), lambda i: (0, 0)),
        ],
        out_specs=[
            pl.BlockSpec((2, RB, TW), lambda i: (0, i, 0)),
            pl.BlockSpec((2, RB, DW), lambda i: (0, i, 0)),
            pl.BlockSpec((2, RB, TW), lambda i: (0, i, 0)),
            pl.BlockSpec((2, RB, DW), lambda i: (0, i, 0)),
        ],
        out_shape=[
            jax.ShapeDtypeStruct((2, NODE_PAD, TW), jnp.float32),
            jax.ShapeDtypeStruct((2, NODE_PAD, DW), jnp.float32),
            jax.ShapeDtypeStruct((2, NODE_PAD, TW), jnp.float32),
            jax.ShapeDtypeStruct((2, NODE_PAD, DW), jnp.float32),
        ],
    )(x, W1, A1s, A1d)


def _pack2(a0, a1, b1, W2, A2s, A2d):
    def body(a0_ref, a1_ref, b_ref, w_ref, as_ref, ad_ref,
             t0_ref, d0_ref, t1_ref, d1_ref):
        parts = []
        for hd in range(4):
            c, p = hd // 2, hd % 2
            a_ref = (a0_ref, a1_ref)[p]
            parts.append(a_ref[c][:, 0:16] / (a_ref[c][:, 16:17] + _EPS))
        h = jnp.concatenate(parts, axis=1) + b_ref[...]
        h = jnp.where(h > 0, h, jnp.exp(h) - 1.0)
        h2 = jnp.dot(h, w_ref[...], preferred_element_type=jnp.float32)
        ss = jnp.dot(h2, as_ref[...], preferred_element_type=jnp.float32)
        sd = jnp.dot(h2, ad_ref[...], preferred_element_type=jnp.float32)
        zt = jnp.zeros((RB, TW - 17), jnp.float32)
        zd = jnp.zeros((RB, DW - 1), jnp.float32)
        for p, (t_ref, d_ref) in ((0, (t0_ref, d0_ref)), (1, (t1_ref, d1_ref))):
            for c in range(2):
                q = 2 * c + p
                t_ref[c] = jnp.concatenate(
                    [h2[:, q * 16:(q + 1) * 16], ss, zt], axis=1)
                d_ref[c] = jnp.concatenate([sd, zd], axis=1)

    return pl.pallas_call(
        body,
        grid=(GRID_R,),
        in_specs=[
            pl.BlockSpec((2, RB, AW), lambda i: (0, i, 0)),
            pl.BlockSpec((2, RB, AW), lambda i: (0, i, 0)),
            pl.BlockSpec((1, HID), lambda i: (0, 0)),
            pl.BlockSpec((HID, HID), lambda i: (0, 0)),
            pl.BlockSpec((HID, 1), lambda i: (0, 0)),
            pl.BlockSpec((HID, 1), lambda i: (0, 0)),
        ],
        out_specs=[
            pl.BlockSpec((2, RB, TW), lambda i: (0, i, 0)),
            pl.BlockSpec((2, RB, DW), lambda i: (0, i, 0)),
            pl.BlockSpec((2, RB, TW), lambda i: (0, i, 0)),
            pl.BlockSpec((2, RB, DW), lambda i: (0, i, 0)),
        ],
        out_shape=[
            jax.ShapeDtypeStruct((2, NODE_PAD, TW), jnp.float32),
            jax.ShapeDtypeStruct((2, NODE_PAD, DW), jnp.float32),
            jax.ShapeDtypeStruct((2, NODE_PAD, TW), jnp.float32),
            jax.ShapeDtypeStruct((2, NODE_PAD, DW), jnp.float32),
        ],
    )(a0, a1, b1, W2, A2s, A2d)


def _pool_mlp(a0, a1, b2, batch3d, Wc1, bc1, Wc2, bc2):
    def body(a0_ref, a1_ref, b_ref, bt_ref, wc1_ref, bc1_ref, wc2_ref,
             bc2_ref, logits_ref, hpool_ref, sums_scr):
        i = pl.program_id(0)

        @pl.when(i == 0)
        def _():
            sums_scr[...] = jnp.zeros((N_GRAPHS, 128), jnp.float32)

        parts = []
        for q in range(4):
            c, p = q // 2, q % 2
            a_ref = (a0_ref, a1_ref)[p]
            parts.append(a_ref[c][:, 0:16] / (a_ref[c][:, 16:17] + _EPS))
        h = jnp.concatenate(parts, axis=1) + b_ref[...]
        h = jnp.where(h > 0, h, jnp.exp(h) - 1.0)
        bt = bt_ref[0, 0]
        p_oh = (lax.broadcasted_iota(jnp.int32, (N_GRAPHS, RB), 0)
                == bt[None, :]).astype(jnp.float32)
        aug = jnp.concatenate(
            [h, jnp.ones((RB, 1), jnp.float32),
             jnp.zeros((RB, 128 - HID - 1), jnp.float32)], axis=1)
        sums_scr[...] += jnp.dot(p_oh, aug, preferred_element_type=jnp.float32)

        @pl.when(i == GRID_R - 1)
        def _():
            s = sums_scr[...]
            cnt = jnp.clip(s[:, HID:HID + 1], 1.0, None)
            hp = s[:, :HID] / cnt
            hpool_ref[...] = hp
            hid = jnp.maximum(
                jnp.dot(hp, wc1_ref[...], preferred_element_type=jnp.float32)
                + bc1_ref[...], 0.0)
            logits_ref[...] = jnp.dot(
                hid, wc2_ref[...], preferred_element_type=jnp.float32) \
                + bc2_ref[...]

    return pl.pallas_call(
        body,
        grid=(GRID_R,),
        in_specs=[
            pl.BlockSpec((2, RB, AW), lambda i: (0, i, 0)),
            pl.BlockSpec((2, RB, AW), lambda i: (0, i, 0)),
            pl.BlockSpec((1, HID), lambda i: (0, 0)),
            pl.BlockSpec((1, 1, RB), lambda i: (i, 0, 0)),
            pl.BlockSpec((HID, HID // 2), lambda i: (0, 0)),
            pl.BlockSpec((1, HID // 2), lambda i: (0, 0)),
            pl.BlockSpec((HID // 2, 1), lambda i: (0, 0)),
            pl.BlockSpec((1, 1), lambda i: (0, 0)),
        ],
        out_specs=[
            pl.BlockSpec((N_GRAPHS, 1), lambda i: (0, 0)),
            pl.BlockSpec((N_GRAPHS, HID), lambda i: (0, 0)),
        ],
        out_shape=[
            jax.ShapeDtypeStruct((N_GRAPHS, 1), jnp.float32),
            jax.ShapeDtypeStruct((N_GRAPHS, HID), jnp.float32),
        ],
        scratch_shapes=[pltpu.VMEM((N_GRAPHS, 128), jnp.float32)],
    )(a0, a1, b2, batch3d, Wc1, bc1, Wc2, bc2)


def kernel(x_lex, edge_index, batch, emb, W1, a1_src, a1_dst, b1,
           W2, a2_src, a2_dst, b2, Wc1, bc1, Wc2, bc2):
    i32 = jnp.int32
    f32 = jnp.float32

    xl = jnp.concatenate([
        x_lex.astype(i32),
        (jnp.arange(NODE_PAD - N_NODES, dtype=i32) % 16),
    ])
    src = edge_index[0].astype(i32)
    dst = edge_index[1].astype(i32)
    epad = EDGE_PAD - N_EDGES
    src_pad = jnp.concatenate([src, jnp.arange(epad, dtype=i32) % 16])
    dst_pad = jnp.concatenate(
        [dst, N_NODES + (jnp.arange(epad, dtype=i32) % 128)])
    src2 = jnp.stack([src_pad, src_pad + NODE_PAD])
    dst2 = jnp.stack([dst_pad, dst_pad + NODE_PAD])
    batch3d = jnp.concatenate([
        batch.astype(i32),
        jnp.full((NODE_PAD - N_NODES,), N_GRAPHS, i32),
    ]).reshape(GRID_R, 1, RB)

    eye = jnp.eye(HEADS, dtype=f32)
    A1s = (a1_src.astype(f32)[:, :, None] * eye[:, None, :]).reshape(HID, HEADS)
    A1d = (a1_dst.astype(f32)[:, :, None] * eye[:, None, :]).reshape(HID, HEADS)
    A2s = a2_src.astype(f32).reshape(HID, 1)
    A2d = a2_dst.astype(f32).reshape(HID, 1)

    zacc = jnp.zeros((NODE_PAD, AW), f32)

    x = _emb_gather(emb.astype(f32), xl)
    t10, d10, t11, d11 = _pack1(x, W1.astype(f32), A1s, A1d)
    a10 = _edge_pass(t10.reshape(2 * NODE_PAD, TW),
                     d10.reshape(2 * NODE_PAD, DW), src2, dst2, dst_pad, zacc)
    a11 = _edge_pass(t11.reshape(2 * NODE_PAD, TW),
                     d11.reshape(2 * NODE_PAD, DW), src2, dst2, dst_pad, zacc)
    t20, d20, t21, d21 = _pack2(a10.reshape(2, NODE_PAD, AW),
                                a11.reshape(2, NODE_PAD, AW),
                                b1.astype(f32).reshape(1, HID),
                                W2.astype(f32), A2s, A2d)
    a20 = _edge_pass(t20.reshape(2 * NODE_PAD, TW),
                     d20.reshape(2 * NODE_PAD, DW), src2, dst2, dst_pad, zacc)
    a21 = _edge_pass(t21.reshape(2 * NODE_PAD, TW),
                     d21.reshape(2 * NODE_PAD, DW), src2, dst2, dst_pad, zacc)
    logits, h_pool = _pool_mlp(a20.reshape(2, NODE_PAD, AW),
                               a21.reshape(2, NODE_PAD, AW),
                               b2.astype(f32).reshape(1, HID), batch3d,
                               Wc1.astype(f32),
                               bc1.astype(f32).reshape(1, HID // 2),
                               Wc2.astype(f32),
                               bc2.astype(f32).reshape(1, 1))
    return (logits, h_pool)


# speedup vs baseline: 30.5957x; 2.3551x over previous
import functools

import jax
import jax.numpy as jnp
from jax import lax
from jax.experimental import pallas as pl
from jax.experimental.pallas import tpu as pltpu
from jax.experimental.pallas import tpu_sc as plsc

N_NODES = 50000
N_EDGES = 800000
VOCAB = 100000
EMBED = 64
HID = 64
HEADS = 4
N_GRAPHS = 128

NC, NS, L = 2, 16, 16
NW = NC * NS

RB = 1024
NODE_PAD = 50176
GRID_R = NODE_PAD // RB

CHUNK = 256
EDGES_PER_TILE = 50176
EDGE_PAD = NS * EDGES_PER_TILE
N_CHUNKS = EDGES_PER_TILE // CHUNK

TW = 32
DW = 16
AW = 24

_EPS = 1e-16

_SC_PARAMS = pltpu.CompilerParams(use_tc_tiling_on_sc=False,
                                  needs_layout_passes=False)


def _emb_gather(emb, idx_pad):
    GW = 128
    n_win = NODE_PAD // GW
    per_w = -(-n_win // NW)
    mesh = plsc.VectorSubcoreMesh(core_axis_name="c", subcore_axis_name="s")

    @functools.partial(
        pl.kernel,
        out_type=jax.ShapeDtypeStruct((NODE_PAD, EMBED), jnp.float32),
        mesh=mesh,
        scratch_types=[
            pltpu.VMEM((GW,), jnp.int32),
            pltpu.VMEM((GW, EMBED), jnp.float32),
            pltpu.SemaphoreType.DMA,
        ],
        compiler_params=_SC_PARAMS,
    )
    def k(emb_hbm, idx_hbm, out_hbm, idx_v, rows_v, sem):
        wid = lax.axis_index("s") * NC + lax.axis_index("c")

        @pl.loop(0, per_w)
        def _(i):
            w = wid * per_w + i

            @pl.when(w < n_win)
            def _():
                base = w * GW
                pltpu.sync_copy(idx_hbm.at[pl.ds(base, GW)], idx_v)
                pltpu.async_copy(emb_hbm.at[idx_v], rows_v, sem).wait()
                pltpu.sync_copy(rows_v, out_hbm.at[pl.ds(base, GW)])

    return k(emb, idx_pad)


def _edge_pass(t_all, d_all, src2, dst2, dst_pad, zacc):
    mesh = plsc.VectorSubcoreMesh(core_axis_name="c", subcore_axis_name="s")
    zrows = NODE_PAD // NS

    @functools.partial(
        pl.kernel,
        out_type=jax.ShapeDtypeStruct((2 * NODE_PAD, AW), jnp.float32),
        mesh=mesh,
        scratch_types=[
            pltpu.VMEM((CHUNK,), jnp.int32),
            pltpu.VMEM((CHUNK,), jnp.int32),
            pltpu.VMEM((CHUNK,), jnp.int32),
            pltpu.VMEM((CHUNK, TW), jnp.float32),
            pltpu.VMEM((CHUNK, DW), jnp.float32),
            pltpu.VMEM((CHUNK, AW), jnp.float32),
            pltpu.VMEM_SHARED((NODE_PAD, AW), jnp.float32),
        ],
        compiler_params=_SC_PARAMS,
    )
    def k(t_hbm, d_hbm, src2_hbm, dst2_hbm, dst_hbm, z_hbm, acc_hbm,
          dsti_v, srco_v, dsto_v, trow_v, drow_v, blk_v, acc_sp):
        cid = lax.axis_index("c")
        sid = lax.axis_index("s")
        coff = cid * NODE_PAD

        pltpu.sync_copy(z_hbm.at[pl.ds(sid * zrows, zrows)],
                        acc_sp.at[pl.ds(sid * zrows, zrows)])
        plsc.subcore_barrier()

        @pl.loop(0, N_CHUNKS)
        def _chunk(ci):
            ebase = sid * EDGES_PER_TILE + ci * CHUNK
            pltpu.sync_copy(src2_hbm.at[cid, pl.ds(ebase, CHUNK)], srco_v)
            pltpu.sync_copy(dst2_hbm.at[cid, pl.ds(ebase, CHUNK)], dsto_v)
            pltpu.sync_copy(dst_hbm.at[pl.ds(ebase, CHUNK)], dsti_v)
            pltpu.sync_copy(t_hbm.at[srco_v], trow_v)
            pltpu.sync_copy(d_hbm.at[dsto_v], drow_v)
            for g in range(CHUNK // L):
                rows = lax.iota(jnp.int32, L) + (g * L)
                c16 = jnp.full((L,), 16, jnp.int32)
                c0 = jnp.zeros((L,), jnp.int32)
                s0 = plsc.load_gather(trow_v, [rows, c16]) + \
                     plsc.load_gather(drow_v, [rows, c0])
                ee = jnp.exp(jnp.maximum(s0, 0.2 * s0))
                plsc.store_scatter(blk_v, [rows, c16], ee)
                for j in range(L):
                    e = g * L + j
                    blk_v[e, pl.ds(0, L)] = trow_v[e, pl.ds(0, L)] * ee[j]
            pltpu.sync_copy(blk_v, acc_sp.at[dsti_v], add=True)

        plsc.subcore_barrier()
        obase = coff + sid * zrows
        pltpu.sync_copy(acc_sp.at[pl.ds(sid * zrows, zrows)],
                        acc_hbm.at[pl.ds(obase, zrows)])

    return k(t_all, d_all, src2, dst2, dst_pad, zacc)


def _pack1(x, W1, A1s, A1d):
    def body(x_ref, w_ref, as_ref, ad_ref, t0_ref, d0_ref, t1_ref, d1_ref):
        h = jnp.dot(x_ref[...], w_ref[...], preferred_element_type=jnp.float32)
        ss = jnp.dot(h, as_ref[...], preferred_element_type=jnp.float32)
        sd = jnp.dot(h, ad_ref[...], preferred_element_type=jnp.float32)
        zt = jnp.zeros((RB, TW - 17), jnp.float32)
        zd = jnp.zeros((RB, DW - 1), jnp.float32)
        for p, (t_ref, d_ref) in ((0, (t0_ref, d0_ref)), (1, (t1_ref, d1_ref))):
            for c in range(2):
                hd = 2 * c + p
                t_ref[c] = jnp.concatenate(
                    [h[:, hd * 16:(hd + 1) * 16], ss[:, hd:hd + 1], zt], axis=1)
                d_ref[c] = jnp.concatenate([sd[:, hd:hd + 1], zd], axis=1)

    return pl.pallas_call(
        body,
        grid=(GRID_R,),
        in_specs=[
            pl.BlockSpec((RB, EMBED), lambda i: (i, 0)),
            pl.BlockSpec((EMBED, HID), lambda i: (0, 0)),
            pl.BlockSpec((HID, HEADS), lambda i: (0, 0)),
            pl.BlockSpec((HID, HEADS), lambda i: (0, 0)),
        ],
        out_specs=[
            pl.BlockSpec((2, RB, TW), lambda i: (0, i, 0)),
            pl.BlockSpec((2, RB, DW), lambda i: (0, i, 0)),
            pl.BlockSpec((2, RB, TW), lambda i: (0, i, 0)),
            pl.BlockSpec((2, RB, DW), lambda i: (0, i, 0)),
        ],
        out_shape=[
            jax.ShapeDtypeStruct((2, NODE_PAD, TW), jnp.float32),
            jax.ShapeDtypeStruct((2, NODE_PAD, DW), jnp.float32),
            jax.ShapeDtypeStruct((2, NODE_PAD, TW), jnp.float32),
            jax.ShapeDtypeStruct((2, NODE_PAD, DW), jnp.float32),
        ],
    )(x, W1, A1s, A1d)


def _pack2(a0, a1, b1, W2, A2s, A2d):
    def body(a0_ref, a1_ref, b_ref, w_ref, as_ref, ad_ref,
             t0_ref, d0_ref, t1_ref, d1_ref):
        parts = []
        for hd in range(4):
            c, p = hd // 2, hd % 2
            a_ref = (a0_ref, a1_ref)[p]
            parts.append(a_ref[c][:, 0:16] / (a_ref[c][:, 16:17] + _EPS))
        h = jnp.concatenate(parts, axis=1) + b_ref[...]
        h = jnp.where(h > 0, h, jnp.exp(h) - 1.0)
        h2 = jnp.dot(h, w_ref[...], preferred_element_type=jnp.float32)
        ss = jnp.dot(h2, as_ref[...], preferred_element_type=jnp.float32)
        sd = jnp.dot(h2, ad_ref[...], preferred_element_type=jnp.float32)
        zt = jnp.zeros((RB, TW - 17), jnp.float32)
        zd = jnp.zeros((RB, DW - 1), jnp.float32)
        for p, (t_ref, d_ref) in ((0, (t0_ref, d0_ref)), (1, (t1_ref, d1_ref))):
            for c in range(2):
                q = 2 * c + p
                t_ref[c] = jnp.concatenate(
                    [h2[:, q * 16:(q + 1) * 16], ss, zt], axis=1)
                d_ref[c] = jnp.concatenate([sd, zd], axis=1)

    return pl.pallas_call(
        body,
        grid=(GRID_R,),
        in_specs=[
            pl.BlockSpec((2, RB, AW), lambda i: (0, i, 0)),
            pl.BlockSpec((2, RB, AW), lambda i: (0, i, 0)),
            pl.BlockSpec((1, HID), lambda i: (0, 0)),
            pl.BlockSpec((HID, HID), lambda i: (0, 0)),
            pl.BlockSpec((HID, 1), lambda i: (0, 0)),
            pl.BlockSpec((HID, 1), lambda i: (0, 0)),
        ],
        out_specs=[
            pl.BlockSpec((2, RB, TW), lambda i: (0, i, 0)),
            pl.BlockSpec((2, RB, DW), lambda i: (0, i, 0)),
            pl.BlockSpec((2, RB, TW), lambda i: (0, i, 0)),
            pl.BlockSpec((2, RB, DW), lambda i: (0, i, 0)),
        ],
        out_shape=[
            jax.ShapeDtypeStruct((2, NODE_PAD, TW), jnp.float32),
            jax.ShapeDtypeStruct((2, NODE_PAD, DW), jnp.float32),
            jax.ShapeDtypeStruct((2, NODE_PAD, TW), jnp.float32),
            jax.ShapeDtypeStruct((2, NODE_PAD, DW), jnp.float32),
        ],
    )(a0, a1, b1, W2, A2s, A2d)


def _pool_mlp(a0, a1, b2, batch3d, Wc1, bc1, Wc2, bc2):
    def body(a0_ref, a1_ref, b_ref, bt_ref, wc1_ref, bc1_ref, wc2_ref,
             bc2_ref, logits_ref, hpool_ref, sums_scr):
        i = pl.program_id(0)

        @pl.when(i == 0)
        def _():
            sums_scr[...] = jnp.zeros((N_GRAPHS, 128), jnp.float32)

        parts = []
        for q in range(4):
            c, p = q // 2, q % 2
            a_ref = (a0_ref, a1_ref)[p]
            parts.append(a_ref[c][:, 0:16] / (a_ref[c][:, 16:17] + _EPS))
        h = jnp.concatenate(parts, axis=1) + b_ref[...]
        h = jnp.where(h > 0, h, jnp.exp(h) - 1.0)
        bt = bt_ref[0, 0]
        p_oh = (lax.broadcasted_iota(jnp.int32, (N_GRAPHS, RB), 0)
                == bt[None, :]).astype(jnp.float32)
        aug = jnp.concatenate(
            [h, jnp.ones((RB, 1), jnp.float32),
             jnp.zeros((RB, 128 - HID - 1), jnp.float32)], axis=1)
        sums_scr[...] += jnp.dot(p_oh, aug, preferred_element_type=jnp.float32)

        @pl.when(i == GRID_R - 1)
        def _():
            s = sums_scr[...]
            cnt = jnp.clip(s[:, HID:HID + 1], 1.0, None)
            hp = s[:, :HID] / cnt
            hpool_ref[...] = hp
            hid = jnp.maximum(
                jnp.dot(hp, wc1_ref[...], preferred_element_type=jnp.float32)
                + bc1_ref[...], 0.0)
            logits_ref[...] = jnp.dot(
                hid, wc2_ref[...], preferred_element_type=jnp.float32) \
                + bc2_ref[...]

    return pl.pallas_call(
        body,
        grid=(GRID_R,),
        in_specs=[
            pl.BlockSpec((2, RB, AW), lambda i: (0, i, 0)),
            pl.BlockSpec((2, RB, AW), lambda i: (0, i, 0)),
            pl.BlockSpec((1, HID), lambda i: (0, 0)),
            pl.BlockSpec((1, 1, RB), lambda i: (i, 0, 0)),
            pl.BlockSpec((HID, HID // 2), lambda i: (0, 0)),
            pl.BlockSpec((1, HID // 2), lambda i: (0, 0)),
            pl.BlockSpec((HID // 2, 1), lambda i: (0, 0)),
            pl.BlockSpec((1, 1), lambda i: (0, 0)),
        ],
        out_specs=[
            pl.BlockSpec((N_GRAPHS, 1), lambda i: (0, 0)),
            pl.BlockSpec((N_GRAPHS, HID), lambda i: (0, 0)),
        ],
        out_shape=[
            jax.ShapeDtypeStruct((N_GRAPHS, 1), jnp.float32),
            jax.ShapeDtypeStruct((N_GRAPHS, HID), jnp.float32),
        ],
        scratch_shapes=[pltpu.VMEM((N_GRAPHS, 128), jnp.float32)],
    )(a0, a1, b2, batch3d, Wc1, bc1, Wc2, bc2)


def kernel(x_lex, edge_index, batch, emb, W1, a1_src, a1_dst, b1,
           W2, a2_src, a2_dst, b2, Wc1, bc1, Wc2, bc2):
    i32 = jnp.int32
    f32 = jnp.float32

    xl = jnp.concatenate([
        x_lex.astype(i32),
        (jnp.arange(NODE_PAD - N_NODES, dtype=i32) % 16),
    ])
    src = edge_index[0].astype(i32)
    dst = edge_index[1].astype(i32)
    epad = EDGE_PAD - N_EDGES
    src_pad = jnp.concatenate([src, jnp.arange(epad, dtype=i32) % 16])
    dst_pad = jnp.concatenate(
        [dst, N_NODES + (jnp.arange(epad, dtype=i32) % 128)])
    src2 = jnp.stack([src_pad, src_pad + NODE_PAD])
    dst2 = jnp.stack([dst_pad, dst_pad + NODE_PAD])
    batch3d = jnp.concatenate([
        batch.astype(i32),
        jnp.full((NODE_PAD - N_NODES,), N_GRAPHS, i32),
    ]).reshape(GRID_R, 1, RB)

    eye = jnp.eye(HEADS, dtype=f32)
    A1s = (a1_src.astype(f32)[:, :, None] * eye[:, None, :]).reshape(HID, HEADS)
    A1d = (a1_dst.astype(f32)[:, :, None] * eye[:, None, :]).reshape(HID, HEADS)
    A2s = a2_src.astype(f32).reshape(HID, 1)
    A2d = a2_dst.astype(f32).reshape(HID, 1)

    zacc = jnp.zeros((NODE_PAD, AW), f32)

    x = _emb_gather(emb.astype(f32), xl)
    t10, d10, t11, d11 = _pack1(x, W1.astype(f32), A1s, A1d)
    a10 = _edge_pass(t10.reshape(2 * NODE_PAD, TW),
                     d10.reshape(2 * NODE_PAD, DW), src2, dst2, dst_pad, zacc)
    a11 = _edge_pass(t11.reshape(2 * NODE_PAD, TW),
                     d11.reshape(2 * NODE_PAD, DW), src2, dst2, dst_pad, zacc)
    t20, d20, t21, d21 = _pack2(a10.reshape(2, NODE_PAD, AW),
                                a11.reshape(2, NODE_PAD, AW),
                                b1.astype(f32).reshape(1, HID),
                                W2.astype(f32), A2s, A2d)
    a20 = _edge_pass(t20.reshape(2 * NODE_PAD, TW),
                     d20.reshape(2 * NODE_PAD, DW), src2, dst2, dst_pad, zacc)
    a21 = _edge_pass(t21.reshape(2 * NODE_PAD, TW),
                     d21.reshape(2 * NODE_PAD, DW), src2, dst2, dst_pad, zacc)
    logits, h_pool = _pool_mlp(a20.reshape(2, NODE_PAD, AW),
                               a21.reshape(2, NODE_PAD, AW),
                               b2.astype(f32).reshape(1, HID), batch3d,
                               Wc1.astype(f32),
                               bc1.astype(f32).reshape(1, HID // 2),
                               Wc2.astype(f32),
                               bc2.astype(f32).reshape(1, 1))
    return (logits, h_pool)
